# Initial kernel scaffold; baseline (speedup 1.0000x reference)
import jax
import jax.numpy as jnp
from jax.experimental import pallas as pl


def kernel(x, edge_attr, params, edge_index_bond, edge_index_complete):
    raise NotImplementedError("write your pallas kernel here")


# speedup vs baseline: 1.0712x; 1.0712x over previous
import functools

import jax
import jax.numpy as jnp
from jax.experimental import pallas as pl
from jax.experimental.pallas import tpu as pltpu

_NN = 10000
_EE = 320000
_HH = 128
_AVG_D_LOG = 1.0


def _mm_body(x_ref, w_ref, b_ref, o_ref, *, act):
    y = jnp.dot(x_ref[...], w_ref[...], preferred_element_type=jnp.float32)
    y = y + b_ref[...]
    if act == "relu":
        y = jax.nn.relu(y)
    o_ref[...] = y


def _pallas_mm(x, w, b, act="none", bm=1024):
    m, k = x.shape
    n = w.shape[1]
    mp = ((m + bm - 1) // bm) * bm
    if mp != m:
        x = jnp.pad(x, ((0, mp - m), (0, 0)))
    out = pl.pallas_call(
        functools.partial(_mm_body, act=act),
        grid=(mp // bm,),
        in_specs=[
            pl.BlockSpec((bm, k), lambda i: (i, 0)),
            pl.BlockSpec((k, n), lambda i: (0, 0)),
            pl.BlockSpec((n,), lambda i: (0,)),
        ],
        out_specs=pl.BlockSpec((bm, n), lambda i: (i, 0)),
        out_shape=jax.ShapeDtypeStruct((mp, n), jnp.float32),
    )(x, w, b)
    return out[:m] if mp != m else out


def _edge_gate_body(t_ref, w2_ref, b2_ref, wse_ref, bse_ref, o_ref):
    t = jax.nn.relu(t_ref[...])
    m = jnp.dot(t, w2_ref[...], preferred_element_type=jnp.float32) + b2_ref[...]
    gate = jax.nn.sigmoid(
        jnp.dot(m, wse_ref[...], preferred_element_type=jnp.float32) + bse_ref[...]
    )
    o_ref[...] = m * gate


def _edge_gate(t, w2, b2, wse, bse, bm=2048):
    m, k = t.shape
    n = w2.shape[1]
    mp = ((m + bm - 1) // bm) * bm
    if mp != m:
        t = jnp.pad(t, ((0, mp - m), (0, 0)))
    out = pl.pallas_call(
        _edge_gate_body,
        grid=(mp // bm,),
        in_specs=[
            pl.BlockSpec((bm, k), lambda i: (i, 0)),
            pl.BlockSpec((k, n), lambda i: (0, 0)),
            pl.BlockSpec((n,), lambda i: (0,)),
            pl.BlockSpec((n, 1), lambda i: (0, 0)),
            pl.BlockSpec((1,), lambda i: (0,)),
        ],
        out_specs=pl.BlockSpec((bm, n), lambda i: (i, 0)),
        out_shape=jax.ShapeDtypeStruct((mp, n), jnp.float32),
    )(t, w2, b2, wse, bse)
    return out[:m] if mp != m else out


def _aggregate(msgs, dst, n, deg, degc, mask, logd):
    s = jax.ops.segment_sum(msgs, dst, num_segments=n)
    mean = s / degc
    sq = jax.ops.segment_sum(msgs * msgs, dst, num_segments=n) / degc
    std = jnp.sqrt(jax.nn.relu(sq - mean * mean) + 1e-5)
    mx = jax.ops.segment_max(msgs, dst, num_segments=n)
    mn = jax.ops.segment_min(msgs, dst, num_segments=n)
    mx = jnp.where(mask, mx, 0.0)
    mn = jnp.where(mask, mn, 0.0)
    h4 = jnp.concatenate([mean, mx, mn, std], axis=-1)
    safe_logd = jnp.where(mask, logd, 1.0)
    amp = h4 * (logd / _AVG_D_LOG)
    att = h4 * (_AVG_D_LOG / safe_logd)
    out = jnp.concatenate([h4, amp, att], axis=-1)
    return jnp.where(mask, out, 0.0)


def _degree_stats(dst, n):
    ones = jnp.ones((dst.shape[0],), dtype=jnp.float32)
    deg = jax.ops.segment_sum(ones, dst, num_segments=n)
    degc = jnp.maximum(deg, 1.0)[:, None]
    mask = (deg > 0)[:, None]
    logd = jnp.log(deg + 1.0)[:, None]
    return deg, degc, mask, logd


def kernel(x, edge_attr, params, edge_index_bond, edge_index_complete):
    n = x.shape[0]
    src, dst = edge_index_bond[0], edge_index_bond[1]
    srcc, dstc = edge_index_complete[0], edge_index_complete[1]

    h = _pallas_mm(x, params["node_in"][0][0], params["node_in"][0][1], act="relu")
    eh = _pallas_mm(edge_attr, params["edge_in"][0][0], params["edge_in"][0][1], act="relu")

    degb = _degree_stats(dst, n)
    degc_ = _degree_stats(dstc, n)

    for p in params["layers"]:
        wpre, bpre = p["pretrans"][0]
        ws, wd, we = wpre[:_HH], wpre[_HH : 2 * _HH], wpre[2 * _HH :]
        ab = _pallas_mm(h, jnp.concatenate([ws, wd], axis=1), jnp.zeros((2 * _HH,), jnp.float32))
        a_tab, b_tab = ab[:, :_HH], ab[:, _HH:]
        c_edge = _pallas_mm(eh, we, bpre)
        e = a_tab[src] + b_tab[dst] + c_edge

        (w1, b1), (w2, b2) = p["pretrans_complete"]
        w1s, w1d = w1[:_HH], w1[_HH:]
        pq = _pallas_mm(h, jnp.concatenate([w1s, w1d], axis=1), jnp.zeros((2 * _HH,), jnp.float32))
        p_tab, q_tab = pq[:, :_HH], pq[:, _HH:]
        t = p_tab[srcc] + q_tab[dstc] + b1
        wse, bse = p["soft_edge"]
        ec = _edge_gate(t, w2, b2, wse, bse)

        f_bond = _aggregate(e, dst, n, *degb)
        f_comp = _aggregate(ec, dstc, n, *degc_)
        wpost, bpost = p["posttrans"][0]
        hin = jnp.concatenate([h, f_bond, f_comp], axis=-1)
        h = _pallas_mm(hin, wpost, bpost) + h

    (wo1, bo1), (wo2, bo2) = params["node_out"]
    h = _pallas_mm(h, wo1, bo1, act="relu")
    h = _pallas_mm(h, wo2, bo2)
    g = jnp.concatenate([h.sum(axis=0), h.mean(axis=0), h.max(axis=0)], axis=-1)

    (wr1, br1), (wr2, br2) = params["readout"]
    g = _pallas_mm(g[None, :], wr1, br1, act="relu", bm=8)
    g = _pallas_mm(g, wr2, br2, bm=8)
    return g[0]
